# baseline (speedup 1.0000x reference)
import functools

import jax
import jax.numpy as jnp
from jax import lax
from jax.experimental import pallas as pl
from jax.experimental.pallas import tpu as pltpu
from jax.experimental.pallas import tpu_sc as plsc

N = 10000
E = 320000
D = 128
N_PAD = 10240
NC = 2
NS = 16
NW = NC * NS
RPT = N_PAD // NS
BM = 1000

E_PAD = 327680
EC = 128
EN = E_PAD // NW // EC
NBUF = 2
EG = EN // NBUF

DC = 80
DN = E // NW // DC
DG = DN // NBUF

_mesh = plsc.VectorSubcoreMesh(core_axis_name="c", subcore_axis_name="s")


@functools.partial(
    pl.kernel,
    mesh=_mesh,
    out_type=jax.ShapeDtypeStruct((NC, 2, N_PAD), jnp.float32),
    scratch_types=[
        pltpu.VMEM((DN, DC), jnp.int32),
        pltpu.VMEM((DN, DC), jnp.int32),
        pltpu.VMEM((DC,), jnp.float32),
        pltpu.VMEM_SHARED((N_PAD,), jnp.float32),
        pltpu.VMEM_SHARED((N_PAD,), jnp.float32),
        pltpu.SemaphoreType.DMA((NBUF,)),
        pltpu.SemaphoreType.DMA((NBUF,)),
    ],
)
def _deg_kernel(src_hbm, dst_hbm, ones_hbm, zeros_hbm, out_hbm,
                src_v, dst_v, ones_v, dego_sh, degi_sh, osem, isem):
    cid = lax.axis_index("c")
    sid = lax.axis_index("s")
    wid = sid * NC + cid
    pltpu.sync_copy(src_hbm.at[wid], src_v)
    pltpu.sync_copy(dst_hbm.at[wid], dst_v)
    pltpu.sync_copy(ones_hbm, ones_v)
    pltpu.sync_copy(zeros_hbm, dego_sh.at[pl.ds(sid * RPT, RPT)])
    pltpu.sync_copy(zeros_hbm, degi_sh.at[pl.ds(sid * RPT, RPT)])
    plsc.subcore_barrier()

    def start(j, b):
        pltpu.async_copy(ones_v, dego_sh.at[src_v.at[j]], osem.at[b], add=True)
        pltpu.async_copy(ones_v, degi_sh.at[dst_v.at[j]], isem.at[b], add=True)

    def drain(j, b):
        pltpu.make_async_copy(ones_v, dego_sh.at[src_v.at[j]], osem.at[b]).wait()
        pltpu.make_async_copy(ones_v, degi_sh.at[dst_v.at[j]], isem.at[b]).wait()

    for b in range(NBUF):
        start(b, b)

    def group(g, carry):
        for b in range(NBUF):
            j = g * NBUF + b
            drain(j, b)
            start(j + NBUF, b)
        return carry

    lax.fori_loop(0, DG - 1, group, 0)
    for b in range(NBUF):
        drain((DG - 1) * NBUF + b, b)
    start(DN - 1, 0)
    drain(DN - 1, 0)
    plsc.subcore_barrier()
    pltpu.sync_copy(dego_sh.at[pl.ds(sid * RPT, RPT)],
                    out_hbm.at[cid, 0, pl.ds(sid * RPT, RPT)])
    pltpu.sync_copy(degi_sh.at[pl.ds(sid * RPT, RPT)],
                    out_hbm.at[cid, 1, pl.ds(sid * RPT, RPT)])


@functools.partial(
    pl.kernel,
    mesh=_mesh,
    out_type=jax.ShapeDtypeStruct((NC, N_PAD, D), jnp.float32),
    scratch_types=[
        pltpu.VMEM((3, NBUF, EC), jnp.int32),
        pltpu.VMEM((3, NBUF, EC), jnp.int32),
        pltpu.VMEM((NBUF, EC, D), jnp.float32),
        pltpu.VMEM_SHARED((N_PAD, D), jnp.float32),
        pltpu.SemaphoreType.DMA((NBUF,)),
        pltpu.SemaphoreType.DMA((NBUF,)),
        pltpu.SemaphoreType.DMA((3,)),
    ],
)
def _edge_kernel(h_hbm, src_hbm, dst_hbm, zrows_hbm, out_hbm,
                 srcr_v, dstr_v, rows_v, acc_sh, gsem, ssem, xsem):
    cid = lax.axis_index("c")
    sid = lax.axis_index("s")
    wid = sid * NC + cid
    rbase = sid * RPT

    def load_idx(g, s):
        pltpu.async_copy(src_hbm.at[wid, g], srcr_v.at[s], xsem.at[s])
        pltpu.async_copy(dst_hbm.at[wid, g], dstr_v.at[s], xsem.at[s])

    def wait_idx(g, s):
        pltpu.make_async_copy(src_hbm.at[wid, g], srcr_v.at[s], xsem.at[s]).wait()
        pltpu.make_async_copy(dst_hbm.at[wid, g], dstr_v.at[s], xsem.at[s]).wait()

    load_idx(0, 0)
    load_idx(1, 1)
    for b in range(NBUF):
        pltpu.sync_copy(zrows_hbm, rows_v.at[b])
    for t in range(RPT // EC):
        pltpu.async_copy(rows_v.at[t % NBUF],
                         acc_sh.at[pl.ds(rbase + t * EC, EC)],
                         gsem.at[t % NBUF])
    for t in range(RPT // EC):
        pltpu.make_async_copy(rows_v.at[t % NBUF],
                              acc_sh.at[pl.ds(rbase + t * EC, EC)],
                              gsem.at[t % NBUF]).wait()
    plsc.subcore_barrier()

    wait_idx(0, 0)

    def do_group(g, s_cur, s_nxt, has_next):
        for b in range(NBUF):
            pltpu.async_copy(h_hbm.at[srcr_v.at[s_cur, b]], rows_v.at[b],
                             gsem.at[b]).wait()
            pltpu.sync_copy(rows_v.at[b], acc_sh.at[dstr_v.at[s_cur, b]],
                            add=True)

    def main_group(g, carry):
        s_cur = lax.rem(g, 3)
        s_nxt = lax.rem(g + 1, 3)
        load_idx(g + 2, lax.rem(g + 2, 3))
        wait_idx(g + 1, s_nxt)
        do_group(g, s_cur, s_nxt, True)
        return carry

    lax.fori_loop(0, EG - 2, main_group, 0)
    wait_idx(EG - 1, (EG - 1) % 3)
    do_group(EG - 2, (EG - 2) % 3, (EG - 1) % 3, True)
    do_group(EG - 1, (EG - 1) % 3, 0, False)
    plsc.subcore_barrier()
    for b in range(NBUF):
        pltpu.async_copy(acc_sh.at[pl.ds(rbase + b * EC, EC)], rows_v.at[b],
                         gsem.at[b])
    for t in range(RPT // EC):
        b = t % NBUF
        pltpu.make_async_copy(acc_sh.at[pl.ds(rbase + t * EC, EC)],
                              rows_v.at[b], gsem.at[b]).wait()
        pltpu.async_copy(rows_v.at[b],
                         out_hbm.at[cid, pl.ds(rbase + t * EC, EC)],
                         ssem.at[b])
        pltpu.make_async_copy(rows_v.at[b],
                              out_hbm.at[cid, pl.ds(rbase + t * EC, EC)],
                              ssem.at[b]).wait()
        if t + NBUF < RPT // EC:
            pltpu.async_copy(acc_sh.at[pl.ds(rbase + (t + NBUF) * EC, EC)],
                             rows_v.at[b], gsem.at[b])


def _norm_body(p_ref, out_ref):
    deg = p_ref[0] + p_ref[1] + 1.0
    out_ref[...] = lax.rsqrt(deg)


def _mm_scale_body(x_ref, w_ref, s_ref, o_ref):
    h = jnp.dot(x_ref[...], w_ref[...], preferred_element_type=jnp.float32)
    o_ref[...] = h * s_ref[...]


def _combine_mm_body(p_ref, hp_ref, ni_ref, b_ref, w_ref, no_ref, o_ref):
    agg = p_ref[0] + p_ref[1] + hp_ref[...]
    h = jnp.maximum(agg * ni_ref[...] + b_ref[...], 0.0)
    o_ref[...] = jnp.dot(h, w_ref[...], preferred_element_type=jnp.float32) * no_ref[...]


def _combine_final_body(p_ref, hp_ref, ni_ref, b_ref, o_ref):
    agg = p_ref[0] + p_ref[1] + hp_ref[...]
    o_ref[...] = agg * ni_ref[...] + b_ref[...]


def _norms(deg_p):
    return pl.pallas_call(
        _norm_body,
        out_shape=jax.ShapeDtypeStruct((2, N_PAD), jnp.float32),
    )(deg_p)


def _mm_scale(xv, W, s_col):
    return pl.pallas_call(
        _mm_scale_body,
        grid=(N // BM,),
        in_specs=[
            pl.BlockSpec((BM, D), lambda i: (i, 0)),
            pl.BlockSpec((D, D), lambda i: (0, 0)),
            pl.BlockSpec((BM, 1), lambda i: (i, 0)),
        ],
        out_specs=pl.BlockSpec((BM, D), lambda i: (i, 0)),
        out_shape=jax.ShapeDtypeStruct((N, D), jnp.float32),
    )(xv, W, s_col)


def _combine_mm(part, hp, ni_col, b_row, W, no_col):
    return pl.pallas_call(
        _combine_mm_body,
        grid=(N // BM,),
        in_specs=[
            pl.BlockSpec((NC, BM, D), lambda i: (0, i, 0)),
            pl.BlockSpec((BM, D), lambda i: (i, 0)),
            pl.BlockSpec((BM, 1), lambda i: (i, 0)),
            pl.BlockSpec((1, D), lambda i: (0, 0)),
            pl.BlockSpec((D, D), lambda i: (0, 0)),
            pl.BlockSpec((BM, 1), lambda i: (i, 0)),
        ],
        out_specs=pl.BlockSpec((BM, D), lambda i: (i, 0)),
        out_shape=jax.ShapeDtypeStruct((N, D), jnp.float32),
    )(part, hp, ni_col, b_row, W, no_col)


def _combine_final(part, hp, ni_col, b_row):
    return pl.pallas_call(
        _combine_final_body,
        grid=(N // BM,),
        in_specs=[
            pl.BlockSpec((NC, BM, D), lambda i: (0, i, 0)),
            pl.BlockSpec((BM, D), lambda i: (i, 0)),
            pl.BlockSpec((BM, 1), lambda i: (i, 0)),
            pl.BlockSpec((1, D), lambda i: (0, 0)),
        ],
        out_specs=pl.BlockSpec((BM, D), lambda i: (i, 0)),
        out_shape=jax.ShapeDtypeStruct((N, D), jnp.float32),
    )(part, hp, ni_col, b_row)


def kernel(x, edge_index, W0, b0, W1, b1):
    src = edge_index[0]
    dst = edge_index[1]
    npad = E_PAD - E
    src_d = src.reshape(NW, DN, DC)
    dst_d = dst.reshape(NW, DN, DC)
    ppt = npad // NW
    src_pad = jnp.zeros((NW, ppt), jnp.int32)
    dst_pad = jnp.broadcast_to(N + jnp.arange(ppt, dtype=jnp.int32), (NW, ppt))
    src_e = jnp.concatenate([src.reshape(NW, E // NW), src_pad], axis=1).reshape(NW, EG, NBUF, EC)
    dst_e = jnp.concatenate([dst.reshape(NW, E // NW), dst_pad], axis=1).reshape(NW, EG, NBUF, EC)
    ones_c = jnp.ones((DC,), jnp.float32)
    zeros_r = jnp.zeros((RPT,), jnp.float32)
    zrows = jnp.zeros((EC, D), jnp.float32)

    deg_p = _deg_kernel(src_d, dst_d, ones_c, zeros_r)
    norms = _norms(deg_p)
    no_col = norms[0, :N].reshape(N, 1)
    ni_col = norms[1, :N].reshape(N, 1)

    h0p = _mm_scale(x, W0, no_col)
    part0 = _edge_kernel(h0p, src_e, dst_e, zrows)
    h1p = _combine_mm(part0, h0p, ni_col, b0.reshape(1, D), W1, no_col)
    part1 = _edge_kernel(h1p, src_e, dst_e, zrows)
    return _combine_final(part1, h1p, ni_col, b1.reshape(1, D))


# speedup vs baseline: 1.6009x; 1.6009x over previous
import functools

import jax
import jax.numpy as jnp
from jax import lax
from jax.experimental import pallas as pl
from jax.experimental.pallas import tpu as pltpu
from jax.experimental.pallas import tpu_sc as plsc

N = 10000
E = 320000
D = 128
N_PAD = 10240
NC = 2
NS = 16
NW = NC * NS
RPT = N_PAD // NS
BM = 1000

EC = 120
EN = 84
E_PAD = NW * EN * EC
ZC = 80

DC = 80
DN = E // NW // DC
NBUF = 2
DG = DN // NBUF

_mesh = plsc.VectorSubcoreMesh(core_axis_name="c", subcore_axis_name="s")


@functools.partial(
    pl.kernel,
    mesh=_mesh,
    out_type=jax.ShapeDtypeStruct((NC, 2, N_PAD), jnp.float32),
    scratch_types=[
        pltpu.VMEM((DN, DC), jnp.int32),
        pltpu.VMEM((DN, DC), jnp.int32),
        pltpu.VMEM((DC,), jnp.float32),
        pltpu.VMEM_SHARED((N_PAD,), jnp.float32),
        pltpu.VMEM_SHARED((N_PAD,), jnp.float32),
        pltpu.SemaphoreType.DMA((NBUF,)),
        pltpu.SemaphoreType.DMA((NBUF,)),
    ],
)
def _deg_kernel(src_hbm, dst_hbm, ones_hbm, zeros_hbm, out_hbm,
                src_v, dst_v, ones_v, dego_sh, degi_sh, osem, isem):
    cid = lax.axis_index("c")
    sid = lax.axis_index("s")
    wid = sid * NC + cid
    pltpu.sync_copy(src_hbm.at[wid], src_v)
    pltpu.sync_copy(dst_hbm.at[wid], dst_v)
    pltpu.sync_copy(ones_hbm, ones_v)
    pltpu.sync_copy(zeros_hbm, dego_sh.at[pl.ds(sid * RPT, RPT)])
    pltpu.sync_copy(zeros_hbm, degi_sh.at[pl.ds(sid * RPT, RPT)])
    plsc.subcore_barrier()

    def start(j, b):
        pltpu.async_copy(ones_v, dego_sh.at[src_v.at[j]], osem.at[b], add=True)
        pltpu.async_copy(ones_v, degi_sh.at[dst_v.at[j]], isem.at[b], add=True)

    def drain(j, b):
        pltpu.make_async_copy(ones_v, dego_sh.at[src_v.at[j]], osem.at[b]).wait()
        pltpu.make_async_copy(ones_v, degi_sh.at[dst_v.at[j]], isem.at[b]).wait()

    for b in range(NBUF):
        start(b, b)

    def group(g, carry):
        for b in range(NBUF):
            j = g * NBUF + b
            drain(j, b)
            start(j + NBUF, b)
        return carry

    lax.fori_loop(0, DG - 1, group, 0)
    for b in range(NBUF):
        drain((DG - 1) * NBUF + b, b)
    start(DN - 1, 0)
    drain(DN - 1, 0)
    plsc.subcore_barrier()
    pltpu.sync_copy(dego_sh.at[pl.ds(sid * RPT, RPT)],
                    out_hbm.at[cid, 0, pl.ds(sid * RPT, RPT)])
    pltpu.sync_copy(degi_sh.at[pl.ds(sid * RPT, RPT)],
                    out_hbm.at[cid, 1, pl.ds(sid * RPT, RPT)])


@functools.partial(
    pl.kernel,
    mesh=_mesh,
    out_type=jax.ShapeDtypeStruct((NC, N_PAD, D), jnp.float32),
    scratch_types=[
        pltpu.VMEM((EN, EC), jnp.int32),
        pltpu.VMEM((EN, EC), jnp.int32),
        pltpu.VMEM((EC, D), jnp.float32),
        pltpu.VMEM_SHARED((N_PAD, D), jnp.float32),
        pltpu.SemaphoreType.DMA,
    ],
)
def _edge_kernel(h_hbm, src_hbm, dst_hbm, zrows_hbm, out_hbm,
                 src_v, dst_v, rows_v, acc_sh, gsem):
    cid = lax.axis_index("c")
    sid = lax.axis_index("s")
    wid = sid * NC + cid
    rbase = sid * RPT
    pltpu.sync_copy(src_hbm.at[wid], src_v)
    pltpu.sync_copy(dst_hbm.at[wid], dst_v)
    pltpu.sync_copy(zrows_hbm, rows_v.at[pl.ds(0, ZC)])
    for t in range(RPT // ZC):
        pltpu.sync_copy(rows_v.at[pl.ds(0, ZC)],
                        acc_sh.at[pl.ds(rbase + t * ZC, ZC)])
    plsc.subcore_barrier()

    def chunk(j, carry):
        pltpu.async_copy(h_hbm.at[src_v.at[j]], rows_v, gsem).wait()
        pltpu.sync_copy(rows_v, acc_sh.at[dst_v.at[j]], add=True)
        return carry

    lax.fori_loop(0, EN, chunk, 0)
    plsc.subcore_barrier()
    for t in range(RPT // ZC):
        pltpu.sync_copy(acc_sh.at[pl.ds(rbase + t * ZC, ZC)],
                        rows_v.at[pl.ds(0, ZC)])
        pltpu.sync_copy(rows_v.at[pl.ds(0, ZC)],
                        out_hbm.at[cid, pl.ds(rbase + t * ZC, ZC)])


def _norm_body(p_ref, out_ref):
    deg = p_ref[0] + p_ref[1] + 1.0
    out_ref[...] = lax.rsqrt(deg)


def _mm_scale_body(x_ref, w_ref, s_ref, o_ref):
    h = jnp.dot(x_ref[...], w_ref[...], preferred_element_type=jnp.float32)
    o_ref[...] = h * s_ref[...]


def _combine_mm_body(p_ref, hp_ref, ni_ref, b_ref, w_ref, no_ref, o_ref):
    agg = p_ref[0] + p_ref[1] + hp_ref[...]
    h = jnp.maximum(agg * ni_ref[...] + b_ref[...], 0.0)
    o_ref[...] = jnp.dot(h, w_ref[...], preferred_element_type=jnp.float32) * no_ref[...]


def _combine_final_body(p_ref, hp_ref, ni_ref, b_ref, o_ref):
    agg = p_ref[0] + p_ref[1] + hp_ref[...]
    o_ref[...] = agg * ni_ref[...] + b_ref[...]


def _norms(deg_p):
    return pl.pallas_call(
        _norm_body,
        out_shape=jax.ShapeDtypeStruct((2, N_PAD), jnp.float32),
    )(deg_p)


def _mm_scale(xv, W, s_col):
    return pl.pallas_call(
        _mm_scale_body,
        grid=(N // BM,),
        in_specs=[
            pl.BlockSpec((BM, D), lambda i: (i, 0)),
            pl.BlockSpec((D, D), lambda i: (0, 0)),
            pl.BlockSpec((BM, 1), lambda i: (i, 0)),
        ],
        out_specs=pl.BlockSpec((BM, D), lambda i: (i, 0)),
        out_shape=jax.ShapeDtypeStruct((N, D), jnp.float32),
    )(xv, W, s_col)


def _combine_mm(part, hp, ni_col, b_row, W, no_col):
    return pl.pallas_call(
        _combine_mm_body,
        grid=(N // BM,),
        in_specs=[
            pl.BlockSpec((NC, BM, D), lambda i: (0, i, 0)),
            pl.BlockSpec((BM, D), lambda i: (i, 0)),
            pl.BlockSpec((BM, 1), lambda i: (i, 0)),
            pl.BlockSpec((1, D), lambda i: (0, 0)),
            pl.BlockSpec((D, D), lambda i: (0, 0)),
            pl.BlockSpec((BM, 1), lambda i: (i, 0)),
        ],
        out_specs=pl.BlockSpec((BM, D), lambda i: (i, 0)),
        out_shape=jax.ShapeDtypeStruct((N, D), jnp.float32),
    )(part, hp, ni_col, b_row, W, no_col)


def _combine_final(part, hp, ni_col, b_row):
    return pl.pallas_call(
        _combine_final_body,
        grid=(N // BM,),
        in_specs=[
            pl.BlockSpec((NC, BM, D), lambda i: (0, i, 0)),
            pl.BlockSpec((BM, D), lambda i: (i, 0)),
            pl.BlockSpec((BM, 1), lambda i: (i, 0)),
            pl.BlockSpec((1, D), lambda i: (0, 0)),
        ],
        out_specs=pl.BlockSpec((BM, D), lambda i: (i, 0)),
        out_shape=jax.ShapeDtypeStruct((N, D), jnp.float32),
    )(part, hp, ni_col, b_row)


def kernel(x, edge_index, W0, b0, W1, b1):
    src = edge_index[0]
    dst = edge_index[1]
    src_d = src.reshape(NW, DN, DC)
    dst_d = dst.reshape(NW, DN, DC)
    ppt = (E_PAD - E) // NW
    src_pad = jnp.zeros((NW, ppt), jnp.int32)
    dst_pad = jnp.broadcast_to(N + jnp.arange(ppt, dtype=jnp.int32), (NW, ppt))
    src_e = jnp.concatenate([src.reshape(NW, E // NW), src_pad], axis=1).reshape(NW, EN, EC)
    dst_e = jnp.concatenate([dst.reshape(NW, E // NW), dst_pad], axis=1).reshape(NW, EN, EC)
    ones_c = jnp.ones((DC,), jnp.float32)
    zeros_r = jnp.zeros((RPT,), jnp.float32)
    zrows = jnp.zeros((ZC, D), jnp.float32)

    deg_p = _deg_kernel(src_d, dst_d, ones_c, zeros_r)
    norms = _norms(deg_p)
    no_col = norms[0, :N].reshape(N, 1)
    ni_col = norms[1, :N].reshape(N, 1)

    h0p = _mm_scale(x, W0, no_col)
    part0 = _edge_kernel(h0p, src_e, dst_e, zrows)
    h1p = _combine_mm(part0, h0p, ni_col, b0.reshape(1, D), W1, no_col)
    part1 = _edge_kernel(h1p, src_e, dst_e, zrows)
    return _combine_final(part1, h1p, ni_col, b1.reshape(1, D))
